# baseline (speedup 1.0000x reference)
import functools

import jax
import jax.numpy as jnp
from jax import lax
from jax.experimental import pallas as pl
from jax.experimental.pallas import tpu as pltpu
from jax.experimental.pallas import tpu_sc as plsc

B, P, NUM_CLASSES, K = 8, 1024, 40, 32
RADII = (0.2, 0.3, 0.4)
LEVELS = ((1024, 512, 0.2), (512, 256, 0.3), (256, 128, 0.4))
NEG_INF = float("-inf")
POS_INF = float("inf")


def _fps_level(px, py, pz, S, ps_ref, mind_ref):
    Bb, Pl = px.shape
    h = Pl // 2
    pxf = jnp.concatenate([px[:, :h], px[:, h:]], 0)
    pyf = jnp.concatenate([py[:, :h], py[:, h:]], 0)
    pzf = jnp.concatenate([pz[:, :h], pz[:, h:]], 0)
    iota_f = (lax.broadcasted_iota(jnp.int32, (2 * Bb, h), 1)
              + jnp.where(lax.broadcasted_iota(jnp.int32, (2 * Bb, h), 0)
                          >= Bb, h, 0))
    lane_s = lax.broadcasted_iota(jnp.int32, (Bb, S), 1)
    mind_ref[0:2 * Bb, 0:h] = jnp.full((2 * Bb, h), POS_INF, jnp.float32)

    def body(i, carry):
        cx, cy, cz = carry
        rec = lane_s == i
        ps_ref[0, 0] = jnp.where(rec, cx, ps_ref[0, 0])
        ps_ref[0, 1] = jnp.where(rec, cy, ps_ref[0, 1])
        ps_ref[0, 2] = jnp.where(rec, cz, ps_ref[0, 2])
        cx2 = jnp.concatenate([cx, cx], 0)
        cy2 = jnp.concatenate([cy, cy], 0)
        cz2 = jnp.concatenate([cz, cz], 0)
        dx = pxf - cx2
        dy = pyf - cy2
        dz = pzf - cz2
        d = dx * dx + dy * dy + dz * dz
        mind = jnp.minimum(mind_ref[0:2 * Bb, 0:h], d)
        mind_ref[0:2 * Bb, 0:h] = mind
        mx = jnp.max(mind, axis=1, keepdims=True)
        mxc = jnp.maximum(mx[0:Bb], mx[Bb:2 * Bb])
        mx2 = jnp.concatenate([mxc, mxc], 0)
        cand = jnp.min(jnp.where(mind == mx2, iota_f, Pl), axis=1,
                       keepdims=True)
        idx = jnp.minimum(cand[0:Bb], cand[Bb:2 * Bb])
        idx2 = jnp.concatenate([idx, idx], 0)
        oh = iota_f == idx2
        ex = jnp.sum(jnp.where(oh, pxf, 0.0), axis=1, keepdims=True)
        ey = jnp.sum(jnp.where(oh, pyf, 0.0), axis=1, keepdims=True)
        ez = jnp.sum(jnp.where(oh, pzf, 0.0), axis=1, keepdims=True)
        cx = ex[0:Bb] + ex[Bb:2 * Bb]
        cy = ey[0:Bb] + ey[Bb:2 * Bb]
        cz = ez[0:Bb] + ez[Bb:2 * Bb]
        return (cx, cy, cz)

    lax.fori_loop(0, S, body, (px[:, 0:1], py[:, 0:1], pz[:, 0:1]))


NG = 2
GB = B // NG


def _fps_kernel(planes_ref, ps1_ref, ps2_ref, ps3_ref, mind_ref):
    ps_refs = (ps1_ref, ps2_ref, ps3_ref)
    src = (planes_ref[0, 0], planes_ref[0, 1], planes_ref[0, 2])
    for li, (Pl, S, r) in enumerate(LEVELS):
        px, py, pz = src
        _fps_level(px, py, pz, S, ps_refs[li], mind_ref)
        src = (ps_refs[li][0, 0], ps_refs[li][0, 1], ps_refs[li][0, 2])


def _run_fps(planes):
    outs = [
        jax.ShapeDtypeStruct((NG, 3, GB, 512), jnp.float32),
        jax.ShapeDtypeStruct((NG, 3, GB, 256), jnp.float32),
        jax.ShapeDtypeStruct((NG, 3, GB, 128), jnp.float32),
    ]
    return pl.pallas_call(
        _fps_kernel,
        grid=(NG,),
        in_specs=[pl.BlockSpec((1, 3, GB, P), lambda i: (i, 0, 0, 0))],
        out_specs=[
            pl.BlockSpec((1, 3, GB, 512), lambda i: (i, 0, 0, 0)),
            pl.BlockSpec((1, 3, GB, 256), lambda i: (i, 0, 0, 0)),
            pl.BlockSpec((1, 3, GB, 128), lambda i: (i, 0, 0, 0)),
        ],
        out_shape=outs,
        scratch_shapes=[pltpu.VMEM((2 * GB, P // 2), jnp.float32)],
        compiler_params=pltpu.CompilerParams(
            dimension_semantics=("parallel",)),
    )(planes.reshape(NG, GB, P, 3).transpose(0, 3, 1, 2))


def _ext_kernel(S, Pl, r, qs_ref, pts_ref, n_ref, d2_ref):
    Bb = qs_ref.shape[2]
    px, py, pz = pts_ref[0, 0], pts_ref[0, 1], pts_ref[0, 2]
    sx, sy, sz = qs_ref[0, 0], qs_ref[0, 1], qs_ref[0, 2]
    dx = sx[:, :, None] - px[:, None, :]
    dy = sy[:, :, None] - py[:, None, :]
    dz = sz[:, :, None] - pz[:, None, :]
    d2 = dx * dx + dy * dy + dz * dz
    d2_ref[...] = jnp.where(d2 <= r * r, d2, POS_INF)

    lane_p3 = lax.broadcasted_iota(jnp.int32, (Bb, S, Pl), 2)
    lane_k = lax.broadcasted_iota(jnp.int32, (Bb, S, K), 2)

    RU = 1

    def round_body(kk, idx0):
        d2v = d2_ref[...]
        nacc = n_ref[0]
        for u in range(RU):
            m = jnp.min(d2v, axis=2, keepdims=True)
            idxv = jnp.min(jnp.where(d2v == m, lane_p3, Pl), axis=2,
                           keepdims=True)
            if u == 0:
                idx0 = jnp.where(kk == 0, idxv, idx0)
            idxf = jnp.where(m < POS_INF, idxv, idx0)
            nacc = jnp.where(lane_k == kk * RU + u, idxf, nacc)
            d2v = jnp.where(lane_p3 == idxv, POS_INF, d2v)
        n_ref[0] = nacc
        d2_ref[...] = d2v
        return idx0

    lax.fori_loop(0, K // RU, round_body, jnp.zeros((Bb, S, 1), jnp.int32))


def _run_ext(qs, pts, S, Pl, r):
    return pl.pallas_call(
        functools.partial(_ext_kernel, S, Pl, r),
        grid=(NG,),
        in_specs=[
            pl.BlockSpec((1, 3, GB, S), lambda i: (i, 0, 0, 0)),
            pl.BlockSpec((1, 3, GB, Pl), lambda i: (i, 0, 0, 0)),
        ],
        out_specs=pl.BlockSpec((1, GB, S, K), lambda i: (i, 0, 0, 0)),
        out_shape=jax.ShapeDtypeStruct((NG, GB, S, K), jnp.int32),
        scratch_shapes=[pltpu.VMEM((GB, S, Pl), jnp.float32)],
        compiler_params=pltpu.CompilerParams(
            dimension_semantics=("parallel",)),
    )(qs, pts)


def _table_kernel(C, h_ref, pos_ref, ps_ref, W1_ref, b1_ref, t_ref, q_ref):
    if C == 3:
        hv = h_ref[0]
        t = (hv[:, 0:1] * W1_ref[0:1, :]
             + hv[:, 1:2] * W1_ref[1:2, :]
             + hv[:, 2:3] * W1_ref[2:3, :])
    else:
        t = jnp.dot(h_ref[0], W1_ref[0:C, :],
                    preferred_element_type=jnp.float32)
    pv = pos_ref[0]
    t = t + (pv[:, 0:1] * W1_ref[C:C + 1, :]
             + pv[:, 1:2] * W1_ref[C + 1:C + 2, :]
             + pv[:, 2:3] * W1_ref[C + 2:C + 3, :])
    t_ref[0] = t + b1_ref[0:1, :]
    sv = ps_ref[0]
    q_ref[0] = (sv[:, 0:1] * W1_ref[C:C + 1, :]
                + sv[:, 1:2] * W1_ref[C + 1:C + 2, :]
                + sv[:, 2:3] * W1_ref[C + 2:C + 3, :])


def _run_table(h, pos_s_prev, ps, W1, b1, C, Pl, S, d1):
    return pl.pallas_call(
        functools.partial(_table_kernel, C),
        grid=(B,),
        in_specs=[
            pl.BlockSpec((1, Pl, C), lambda b: (b, 0, 0)),
            pl.BlockSpec((1, Pl, 3), lambda b: (b, 0, 0)),
            pl.BlockSpec((1, S, 3), lambda b: (b, 0, 0)),
            pl.BlockSpec((C + 3, d1), lambda b: (0, 0)),
            pl.BlockSpec((1, d1), lambda b: (0, 0)),
        ],
        out_specs=[
            pl.BlockSpec((1, Pl, d1), lambda b: (b, 0, 0)),
            pl.BlockSpec((1, S, d1), lambda b: (b, 0, 0)),
        ],
        out_shape=[
            jax.ShapeDtypeStruct((B, Pl, d1), jnp.float32),
            jax.ShapeDtypeStruct((B, S, d1), jnp.float32),
        ],
        compiler_params=pltpu.CompilerParams(
            dimension_semantics=("parallel",)),
    )(h, pos_s_prev, ps, W1, b1)


def _sc_gather(table, idx, N, D):
    NC, NS = 2, 16
    NW = NC * NS
    n = N // NW
    CH = 128
    nch = n // CH
    idx2 = idx.reshape(NW * nch, CH)
    mesh = plsc.VectorSubcoreMesh(core_axis_name="c", subcore_axis_name="s")

    UB = min(nch, 8)
    nblk = nch // UB

    @functools.partial(
        pl.kernel,
        out_type=jax.ShapeDtypeStruct((N, D), jnp.float32),
        mesh=mesh,
        scratch_types=[
            pltpu.VMEM((nch, CH), jnp.int32),
            pltpu.VMEM((CH,), jnp.int32),
            pltpu.VMEM((CH,), jnp.int32),
            pltpu.VMEM((CH, D), jnp.float32),
            pltpu.VMEM((CH, D), jnp.float32),
            pltpu.SemaphoreType.DMA,
            pltpu.SemaphoreType.DMA,
            pltpu.SemaphoreType.DMA,
            pltpu.SemaphoreType.DMA,
        ],
    )
    def gk(table_hbm, idx_hbm, out_hbm, idx_all, idx0, idx1, buf0, buf1,
           semg0, semg1, semo0, semo1):
        wid = lax.axis_index("s") * NC + lax.axis_index("c")
        base = wid * n
        pltpu.sync_copy(idx_hbm.at[pl.ds(wid * nch, nch)], idx_all)
        idxs = (idx0, idx1)
        bufs = (buf0, buf1)
        semg = (semg0, semg1)
        semo = (semo0, semo1)

        def block(blk, carry):
            gh = [None, None]
            oh = [None, None]
            for j2 in range(UB):
                p = j2 % 2
                j = blk * UB + j2
                for i in range(CH // 16):
                    idxs[p][pl.ds(i * 16, 16)] = idx_all[j, pl.ds(i * 16, 16)]
                if oh[p] is not None:
                    oh[p].wait()
                gh[p] = pltpu.async_copy(table_hbm.at[idxs[p]], bufs[p],
                                         semg[p])
                if gh[1 - p] is not None:
                    gh[1 - p].wait()
                    oh[1 - p] = pltpu.async_copy(
                        bufs[1 - p],
                        out_hbm.at[pl.ds(base + (j - 1) * CH, CH)],
                        semo[1 - p])
                    gh[1 - p] = None
            pl_last = (UB - 1) % 2
            gh[pl_last].wait()
            oh[pl_last] = pltpu.async_copy(
                bufs[pl_last],
                out_hbm.at[pl.ds(base + (blk * UB + UB - 1) * CH, CH)],
                semo[pl_last])
            for p in range(2):
                if oh[p] is not None:
                    oh[p].wait()
            return carry

        lax.fori_loop(0, nblk, block, 0)

    return gk(table, idx2)


def _agg_kernel(g_ref, q_ref, W2_ref, b2_ref, out_ref):
    qv = q_ref[0]
    S, d1 = qv.shape
    d2 = W2_ref.shape[1]
    m = jnp.maximum(g_ref[0] - qv[None, :, :], 0.0)
    z = jnp.dot(m.reshape(K * S, d1), W2_ref[:, :],
                preferred_element_type=jnp.float32).reshape(K, S, d2)
    zs = [z[k] for k in range(K)]
    while len(zs) > 1:
        zs = [jnp.maximum(zs[i], zs[i + 1]) for i in range(0, len(zs), 2)]
    out_ref[0] = jnp.maximum(zs[0] + b2_ref[0:1, :], 0.0)


def _run_agg(g, q, W2, b2, S, d1, d2):
    return pl.pallas_call(
        _agg_kernel,
        grid=(B,),
        in_specs=[
            pl.BlockSpec((1, K, S, d1), lambda b: (b, 0, 0, 0)),
            pl.BlockSpec((1, S, d1), lambda b: (b, 0, 0)),
            pl.BlockSpec((d1, d2), lambda b: (0, 0)),
            pl.BlockSpec((1, d2), lambda b: (0, 0)),
        ],
        out_specs=pl.BlockSpec((1, S, d2), lambda b: (b, 0, 0)),
        out_shape=jax.ShapeDtypeStruct((B, S, d2), jnp.float32),
        compiler_params=pltpu.CompilerParams(
            dimension_semantics=("parallel",)),
    )(g, q, W2, b2)


def _ga_kernel(h_ref, ps_ref, W1_ref, b1_ref, W2_ref, b2_ref, out_ref):
    C = h_ref.shape[2]
    t = jnp.dot(h_ref[0], W1_ref[0:C, :], preferred_element_type=jnp.float32)
    pv = ps_ref[0]
    t = t + (pv[:, 0:1] * W1_ref[C:C + 1, :]
             + pv[:, 1:2] * W1_ref[C + 1:C + 2, :]
             + pv[:, 2:3] * W1_ref[C + 2:C + 3, :])
    t = jnp.maximum(t + b1_ref[0:1, :], 0.0)
    z = jnp.dot(t, W2_ref[:, :], preferred_element_type=jnp.float32)
    z = jnp.maximum(z + b2_ref[0:1, :], 0.0)
    out_ref[0] = jnp.max(z, axis=0, keepdims=True)


def _run_ga(h3, ps3, W1, b1, W2, b2):
    return pl.pallas_call(
        _ga_kernel,
        grid=(B,),
        in_specs=[
            pl.BlockSpec((1, 128, 256), lambda b: (b, 0, 0)),
            pl.BlockSpec((1, 128, 3), lambda b: (b, 0, 0)),
            pl.BlockSpec((259, 512), lambda b: (0, 0)),
            pl.BlockSpec((1, 512), lambda b: (0, 0)),
            pl.BlockSpec((512, 1024), lambda b: (0, 0)),
            pl.BlockSpec((1, 1024), lambda b: (0, 0)),
        ],
        out_specs=pl.BlockSpec((1, 1, 1024), lambda b: (b, 0, 0)),
        out_shape=jax.ShapeDtypeStruct((B, 1, 1024), jnp.float32),
        compiler_params=pltpu.CompilerParams(
            dimension_semantics=("parallel",)),
    )(h3, ps3, W1, b1, W2, b2).reshape(B, 1024)


def _head_kernel(g_ref, W1_ref, b1_ref, W2_ref, b2_ref, W3_ref, b3_ref,
                 out_ref):
    h = jnp.dot(g_ref[:, :], W1_ref[:, :], preferred_element_type=jnp.float32)
    h = jnp.maximum(h + b1_ref[0:1, :], 0.0)
    h = jnp.dot(h, W2_ref[:, :], preferred_element_type=jnp.float32)
    h = jnp.maximum(h + b2_ref[0:1, :], 0.0)
    h = jnp.dot(h, W3_ref[:, :], preferred_element_type=jnp.float32)
    out_ref[:, :] = h + b3_ref[0:1, :]


def _run_head(g, lin1_W, lin1_b, lin2_W, lin2_b, lin3_W, lin3_b):
    return pl.pallas_call(
        _head_kernel,
        out_shape=jax.ShapeDtypeStruct((B, NUM_CLASSES), jnp.float32),
    )(g, lin1_W, lin1_b, lin2_W, lin2_b, lin3_W, lin3_b)


def kernel(x, pos, batch, sa1_W1, sa1_b1, sa1_W2, sa1_b2, sa2_W1, sa2_b1,
           sa2_W2, sa2_b2, sa3_W1, sa3_b1, sa3_W2, sa3_b2, ga_W1, ga_b1,
           ga_W2, ga_b2, lin1_W, lin1_b, lin2_W, lin2_b, lin3_W, lin3_b):
    pos_bp3 = pos.reshape(B, P, 3)
    pos4 = pos_bp3.reshape(NG, GB, P, 3).transpose(0, 3, 1, 2)
    ps1, ps2, ps3 = _run_fps(pos_bp3)

    ps_mats = [p.transpose(0, 2, 3, 1).reshape(B, -1, 3)
               for p in (ps1, ps2, ps3)]

    h = x.reshape(B, P, 3)
    pos_prev = pos_bp3
    level_ws = (
        (jnp.pad(sa1_W1, ((0, 0), (0, 96))), jnp.pad(sa1_b1, (0, 96)),
         jnp.pad(sa1_W2, ((0, 96), (0, 0))), sa1_b2),
        (sa2_W1, sa2_b1, sa2_W2, sa2_b2),
        (sa3_W1, sa3_b1, sa3_W2, sa3_b2),
    )
    qs_list = (ps1, ps2, ps3)
    pts_list = (pos4, ps1, ps2)
    nidx = _run_ext(qs_list[0], pts_list[0], LEVELS[0][1], LEVELS[0][0],
                    LEVELS[0][2])
    for li, (Pl, S, _r) in enumerate(LEVELS):
        W1, b1, W2, b2 = level_ws[li]
        C = h.shape[2]
        d1 = W1.shape[1]
        d2 = W2.shape[1]
        t, q = _run_table(h, pos_prev, ps_mats[li], W1, b1.reshape(1, d1),
                          C, Pl, S, d1)
        gidx = (jnp.transpose(nidx.reshape(B, S, K), (0, 2, 1))
                + (jnp.arange(B, dtype=jnp.int32) * Pl)[:, None, None])
        N = B * K * S
        g = _sc_gather(t.reshape(B * Pl, d1), gidx.reshape(-1), N, d1)
        if li + 1 < len(LEVELS):
            nl = LEVELS[li + 1]
            nidx = _run_ext(qs_list[li + 1], pts_list[li + 1], nl[1], nl[0],
                            nl[2])
        h = _run_agg(g.reshape(B, K, S, d1), q, W2, b2.reshape(1, d2),
                     S, d1, d2)
        pos_prev = ps_mats[li]

    g = _run_ga(h, ps_mats[2], ga_W1, ga_b1.reshape(1, 512),
                ga_W2, ga_b2.reshape(1, 1024))
    return _run_head(g, lin1_W, lin1_b.reshape(1, 512),
                     lin2_W, lin2_b.reshape(1, 256),
                     lin3_W, lin3_b.reshape(1, 40))


# speedup vs baseline: 1.2332x; 1.2332x over previous
import functools

import jax
import jax.numpy as jnp
from jax import lax
from jax.experimental import pallas as pl
from jax.experimental.pallas import tpu as pltpu
from jax.experimental.pallas import tpu_sc as plsc

B, P, NUM_CLASSES, K = 8, 1024, 40, 32
RADII = (0.2, 0.3, 0.4)
LEVELS = ((1024, 512, 0.2), (512, 256, 0.3), (256, 128, 0.4))
NEG_INF = float("-inf")
POS_INF = float("inf")


def _fps_planes(px, py, pz, S):
    Bb, Pl = px.shape
    lane_s = lax.broadcasted_iota(jnp.int32, (Bb, S), 1)
    lane_p = lax.broadcasted_iota(jnp.int32, (Bb, Pl), 1)

    def body(i, carry):
        mind, cx, cy, cz, sx, sy, sz = carry
        rec = lane_s == i
        sx = jnp.where(rec, cx, sx)
        sy = jnp.where(rec, cy, sy)
        sz = jnp.where(rec, cz, sz)
        dx = px - cx
        dy = py - cy
        dz = pz - cz
        d = dx * dx + dy * dy + dz * dz
        mind = jnp.minimum(mind, d)
        mx = jnp.max(mind, axis=1, keepdims=True)
        idx = jnp.min(jnp.where(mind == mx, lane_p, Pl), axis=1, keepdims=True)
        oh = lane_p == idx
        cx = jnp.sum(jnp.where(oh, px, 0.0), axis=1, keepdims=True)
        cy = jnp.sum(jnp.where(oh, py, 0.0), axis=1, keepdims=True)
        cz = jnp.sum(jnp.where(oh, pz, 0.0), axis=1, keepdims=True)
        return (mind, cx, cy, cz, sx, sy, sz)

    init = (
        jnp.full((Bb, Pl), POS_INF, jnp.float32),
        px[:, 0:1], py[:, 0:1], pz[:, 0:1],
        jnp.zeros((Bb, S), jnp.float32),
        jnp.zeros((Bb, S), jnp.float32),
        jnp.zeros((Bb, S), jnp.float32),
    )
    _, _, _, _, sx, sy, sz = lax.fori_loop(0, S, body, init)
    return sx, sy, sz


NG = 2
GB = B // NG


def _fps_kernel(planes_ref, ps1_ref, ps2_ref, ps3_ref):
    ps_refs = (ps1_ref, ps2_ref, ps3_ref)
    src = (planes_ref[0, 0], planes_ref[0, 1], planes_ref[0, 2])
    for li, (Pl, S, r) in enumerate(LEVELS):
        px, py, pz = src
        sx, sy, sz = _fps_planes(px, py, pz, S)
        ps_refs[li][0, 0] = sx
        ps_refs[li][0, 1] = sy
        ps_refs[li][0, 2] = sz
        src = (sx, sy, sz)


def _run_fps(planes):
    outs = [
        jax.ShapeDtypeStruct((NG, 3, GB, 512), jnp.float32),
        jax.ShapeDtypeStruct((NG, 3, GB, 256), jnp.float32),
        jax.ShapeDtypeStruct((NG, 3, GB, 128), jnp.float32),
    ]
    return pl.pallas_call(
        _fps_kernel,
        grid=(NG,),
        in_specs=[pl.BlockSpec((1, 3, GB, P), lambda i: (i, 0, 0, 0))],
        out_specs=[
            pl.BlockSpec((1, 3, GB, 512), lambda i: (i, 0, 0, 0)),
            pl.BlockSpec((1, 3, GB, 256), lambda i: (i, 0, 0, 0)),
            pl.BlockSpec((1, 3, GB, 128), lambda i: (i, 0, 0, 0)),
        ],
        out_shape=outs,
        compiler_params=pltpu.CompilerParams(
            dimension_semantics=("parallel",)),
    )(planes.reshape(NG, GB, P, 3).transpose(0, 3, 1, 2))


def _ext_kernel(S, Pl, r, qs_ref, pts_ref, n_ref, d2_ref):
    Bb = qs_ref.shape[2]
    px, py, pz = pts_ref[0, 0], pts_ref[0, 1], pts_ref[0, 2]
    sx, sy, sz = qs_ref[0, 0], qs_ref[0, 1], qs_ref[0, 2]
    dx = sx[:, :, None] - px[:, None, :]
    dy = sy[:, :, None] - py[:, None, :]
    dz = sz[:, :, None] - pz[:, None, :]
    d2 = dx * dx + dy * dy + dz * dz
    d2_ref[...] = jnp.where(d2 <= r * r, d2, POS_INF)

    lane_p3 = lax.broadcasted_iota(jnp.int32, (Bb, S, Pl), 2)
    lane_k = lax.broadcasted_iota(jnp.int32, (Bb, S, K), 2)

    RU = 4

    def round_body(kk, idx0):
        d2v = d2_ref[...]
        nacc = n_ref[0]
        for u in range(RU):
            m = jnp.min(d2v, axis=2, keepdims=True)
            idxv = jnp.min(jnp.where(d2v == m, lane_p3, Pl), axis=2,
                           keepdims=True)
            if u == 0:
                idx0 = jnp.where(kk == 0, idxv, idx0)
            idxf = jnp.where(m < POS_INF, idxv, idx0)
            nacc = jnp.where(lane_k == kk * RU + u, idxf, nacc)
            d2v = jnp.where(lane_p3 == idxv, POS_INF, d2v)
        n_ref[0] = nacc
        d2_ref[...] = d2v
        return idx0

    lax.fori_loop(0, K // RU, round_body, jnp.zeros((Bb, S, 1), jnp.int32))


def _run_ext(qs, pts, S, Pl, r):
    return pl.pallas_call(
        functools.partial(_ext_kernel, S, Pl, r),
        grid=(NG,),
        in_specs=[
            pl.BlockSpec((1, 3, GB, S), lambda i: (i, 0, 0, 0)),
            pl.BlockSpec((1, 3, GB, Pl), lambda i: (i, 0, 0, 0)),
        ],
        out_specs=pl.BlockSpec((1, GB, S, K), lambda i: (i, 0, 0, 0)),
        out_shape=jax.ShapeDtypeStruct((NG, GB, S, K), jnp.int32),
        scratch_shapes=[pltpu.VMEM((GB, S, Pl), jnp.float32)],
        compiler_params=pltpu.CompilerParams(
            dimension_semantics=("parallel",)),
    )(qs, pts)


def _table_kernel(C, h_ref, pos_ref, ps_ref, W1_ref, b1_ref, t_ref, q_ref):
    if C == 3:
        hv = h_ref[0]
        t = (hv[:, 0:1] * W1_ref[0:1, :]
             + hv[:, 1:2] * W1_ref[1:2, :]
             + hv[:, 2:3] * W1_ref[2:3, :])
    else:
        t = jnp.dot(h_ref[0], W1_ref[0:C, :],
                    preferred_element_type=jnp.float32)
    pv = pos_ref[0]
    t = t + (pv[:, 0:1] * W1_ref[C:C + 1, :]
             + pv[:, 1:2] * W1_ref[C + 1:C + 2, :]
             + pv[:, 2:3] * W1_ref[C + 2:C + 3, :])
    t_ref[0] = t + b1_ref[0:1, :]
    sv = ps_ref[0]
    q_ref[0] = (sv[:, 0:1] * W1_ref[C:C + 1, :]
                + sv[:, 1:2] * W1_ref[C + 1:C + 2, :]
                + sv[:, 2:3] * W1_ref[C + 2:C + 3, :])


def _run_table(h, pos_s_prev, ps, W1, b1, C, Pl, S, d1):
    return pl.pallas_call(
        functools.partial(_table_kernel, C),
        grid=(B,),
        in_specs=[
            pl.BlockSpec((1, Pl, C), lambda b: (b, 0, 0)),
            pl.BlockSpec((1, Pl, 3), lambda b: (b, 0, 0)),
            pl.BlockSpec((1, S, 3), lambda b: (b, 0, 0)),
            pl.BlockSpec((C + 3, d1), lambda b: (0, 0)),
            pl.BlockSpec((1, d1), lambda b: (0, 0)),
        ],
        out_specs=[
            pl.BlockSpec((1, Pl, d1), lambda b: (b, 0, 0)),
            pl.BlockSpec((1, S, d1), lambda b: (b, 0, 0)),
        ],
        out_shape=[
            jax.ShapeDtypeStruct((B, Pl, d1), jnp.float32),
            jax.ShapeDtypeStruct((B, S, d1), jnp.float32),
        ],
        compiler_params=pltpu.CompilerParams(
            dimension_semantics=("parallel",)),
    )(h, pos_s_prev, ps, W1, b1)


def _sc_gather(table, idx, N, D):
    NC, NS = 2, 16
    NW = NC * NS
    n = N // NW
    CH = 128
    nch = n // CH
    idx2 = idx.reshape(NW * nch, CH)
    mesh = plsc.VectorSubcoreMesh(core_axis_name="c", subcore_axis_name="s")

    UB = min(nch, 8)
    nblk = nch // UB

    @functools.partial(
        pl.kernel,
        out_type=jax.ShapeDtypeStruct((N, D), jnp.float32),
        mesh=mesh,
        scratch_types=[
            pltpu.VMEM((nch, CH), jnp.int32),
            pltpu.VMEM((CH,), jnp.int32),
            pltpu.VMEM((CH,), jnp.int32),
            pltpu.VMEM((CH, D), jnp.float32),
            pltpu.VMEM((CH, D), jnp.float32),
            pltpu.SemaphoreType.DMA,
            pltpu.SemaphoreType.DMA,
            pltpu.SemaphoreType.DMA,
            pltpu.SemaphoreType.DMA,
        ],
    )
    def gk(table_hbm, idx_hbm, out_hbm, idx_all, idx0, idx1, buf0, buf1,
           semg0, semg1, semo0, semo1):
        wid = lax.axis_index("s") * NC + lax.axis_index("c")
        base = wid * n
        pltpu.sync_copy(idx_hbm.at[pl.ds(wid * nch, nch)], idx_all)
        idxs = (idx0, idx1)
        bufs = (buf0, buf1)
        semg = (semg0, semg1)
        semo = (semo0, semo1)

        def block(blk, carry):
            gh = [None, None]
            oh = [None, None]
            for j2 in range(UB):
                p = j2 % 2
                j = blk * UB + j2
                for i in range(CH // 16):
                    idxs[p][pl.ds(i * 16, 16)] = idx_all[j, pl.ds(i * 16, 16)]
                if oh[p] is not None:
                    oh[p].wait()
                gh[p] = pltpu.async_copy(table_hbm.at[idxs[p]], bufs[p],
                                         semg[p])
                if gh[1 - p] is not None:
                    gh[1 - p].wait()
                    oh[1 - p] = pltpu.async_copy(
                        bufs[1 - p],
                        out_hbm.at[pl.ds(base + (j - 1) * CH, CH)],
                        semo[1 - p])
                    gh[1 - p] = None
            pl_last = (UB - 1) % 2
            gh[pl_last].wait()
            oh[pl_last] = pltpu.async_copy(
                bufs[pl_last],
                out_hbm.at[pl.ds(base + (blk * UB + UB - 1) * CH, CH)],
                semo[pl_last])
            for p in range(2):
                if oh[p] is not None:
                    oh[p].wait()
            return carry

        lax.fori_loop(0, nblk, block, 0)

    return gk(table, idx2)


def _agg_kernel(g_ref, q_ref, W2_ref, b2_ref, out_ref):
    qv = q_ref[0]
    S, d1 = qv.shape
    d2 = W2_ref.shape[1]
    m = jnp.maximum(g_ref[0] - qv[None, :, :], 0.0)
    z = jnp.dot(m.reshape(K * S, d1), W2_ref[:, :],
                preferred_element_type=jnp.float32).reshape(K, S, d2)
    zs = [z[k] for k in range(K)]
    while len(zs) > 1:
        zs = [jnp.maximum(zs[i], zs[i + 1]) for i in range(0, len(zs), 2)]
    out_ref[0] = jnp.maximum(zs[0] + b2_ref[0:1, :], 0.0)


def _run_agg(g, q, W2, b2, S, d1, d2):
    return pl.pallas_call(
        _agg_kernel,
        grid=(B,),
        in_specs=[
            pl.BlockSpec((1, K, S, d1), lambda b: (b, 0, 0, 0)),
            pl.BlockSpec((1, S, d1), lambda b: (b, 0, 0)),
            pl.BlockSpec((d1, d2), lambda b: (0, 0)),
            pl.BlockSpec((1, d2), lambda b: (0, 0)),
        ],
        out_specs=pl.BlockSpec((1, S, d2), lambda b: (b, 0, 0)),
        out_shape=jax.ShapeDtypeStruct((B, S, d2), jnp.float32),
        compiler_params=pltpu.CompilerParams(
            dimension_semantics=("parallel",)),
    )(g, q, W2, b2)


def _ga_kernel(h_ref, ps_ref, W1_ref, b1_ref, W2_ref, b2_ref, out_ref):
    C = h_ref.shape[2]
    t = jnp.dot(h_ref[0], W1_ref[0:C, :], preferred_element_type=jnp.float32)
    pv = ps_ref[0]
    t = t + (pv[:, 0:1] * W1_ref[C:C + 1, :]
             + pv[:, 1:2] * W1_ref[C + 1:C + 2, :]
             + pv[:, 2:3] * W1_ref[C + 2:C + 3, :])
    t = jnp.maximum(t + b1_ref[0:1, :], 0.0)
    z = jnp.dot(t, W2_ref[:, :], preferred_element_type=jnp.float32)
    z = jnp.maximum(z + b2_ref[0:1, :], 0.0)
    out_ref[0] = jnp.max(z, axis=0, keepdims=True)


def _run_ga(h3, ps3, W1, b1, W2, b2):
    return pl.pallas_call(
        _ga_kernel,
        grid=(B,),
        in_specs=[
            pl.BlockSpec((1, 128, 256), lambda b: (b, 0, 0)),
            pl.BlockSpec((1, 128, 3), lambda b: (b, 0, 0)),
            pl.BlockSpec((259, 512), lambda b: (0, 0)),
            pl.BlockSpec((1, 512), lambda b: (0, 0)),
            pl.BlockSpec((512, 1024), lambda b: (0, 0)),
            pl.BlockSpec((1, 1024), lambda b: (0, 0)),
        ],
        out_specs=pl.BlockSpec((1, 1, 1024), lambda b: (b, 0, 0)),
        out_shape=jax.ShapeDtypeStruct((B, 1, 1024), jnp.float32),
        compiler_params=pltpu.CompilerParams(
            dimension_semantics=("parallel",)),
    )(h3, ps3, W1, b1, W2, b2).reshape(B, 1024)


def _head_kernel(g_ref, W1_ref, b1_ref, W2_ref, b2_ref, W3_ref, b3_ref,
                 out_ref):
    h = jnp.dot(g_ref[:, :], W1_ref[:, :], preferred_element_type=jnp.float32)
    h = jnp.maximum(h + b1_ref[0:1, :], 0.0)
    h = jnp.dot(h, W2_ref[:, :], preferred_element_type=jnp.float32)
    h = jnp.maximum(h + b2_ref[0:1, :], 0.0)
    h = jnp.dot(h, W3_ref[:, :], preferred_element_type=jnp.float32)
    out_ref[:, :] = h + b3_ref[0:1, :]


def _run_head(g, lin1_W, lin1_b, lin2_W, lin2_b, lin3_W, lin3_b):
    return pl.pallas_call(
        _head_kernel,
        out_shape=jax.ShapeDtypeStruct((B, NUM_CLASSES), jnp.float32),
    )(g, lin1_W, lin1_b, lin2_W, lin2_b, lin3_W, lin3_b)


def kernel(x, pos, batch, sa1_W1, sa1_b1, sa1_W2, sa1_b2, sa2_W1, sa2_b1,
           sa2_W2, sa2_b2, sa3_W1, sa3_b1, sa3_W2, sa3_b2, ga_W1, ga_b1,
           ga_W2, ga_b2, lin1_W, lin1_b, lin2_W, lin2_b, lin3_W, lin3_b):
    pos_bp3 = pos.reshape(B, P, 3)
    pos4 = pos_bp3.reshape(NG, GB, P, 3).transpose(0, 3, 1, 2)
    ps1, ps2, ps3 = _run_fps(pos_bp3)

    ps_mats = [p.transpose(0, 2, 3, 1).reshape(B, -1, 3)
               for p in (ps1, ps2, ps3)]

    h = x.reshape(B, P, 3)
    pos_prev = pos_bp3
    level_ws = (
        (jnp.pad(sa1_W1, ((0, 0), (0, 96))), jnp.pad(sa1_b1, (0, 96)),
         jnp.pad(sa1_W2, ((0, 96), (0, 0))), sa1_b2),
        (sa2_W1, sa2_b1, sa2_W2, sa2_b2),
        (sa3_W1, sa3_b1, sa3_W2, sa3_b2),
    )
    qs_list = (ps1, ps2, ps3)
    pts_list = (pos4, ps1, ps2)
    nidx = _run_ext(qs_list[0], pts_list[0], LEVELS[0][1], LEVELS[0][0],
                    LEVELS[0][2])
    for li, (Pl, S, _r) in enumerate(LEVELS):
        W1, b1, W2, b2 = level_ws[li]
        C = h.shape[2]
        d1 = W1.shape[1]
        d2 = W2.shape[1]
        t, q = _run_table(h, pos_prev, ps_mats[li], W1, b1.reshape(1, d1),
                          C, Pl, S, d1)
        gidx = (jnp.transpose(nidx.reshape(B, S, K), (0, 2, 1))
                + (jnp.arange(B, dtype=jnp.int32) * Pl)[:, None, None])
        N = B * K * S
        g = _sc_gather(t.reshape(B * Pl, d1), gidx.reshape(-1), N, d1)
        if li + 1 < len(LEVELS):
            nl = LEVELS[li + 1]
            nidx = _run_ext(qs_list[li + 1], pts_list[li + 1], nl[1], nl[0],
                            nl[2])
        h = _run_agg(g.reshape(B, K, S, d1), q, W2, b2.reshape(1, d2),
                     S, d1, d2)
        pos_prev = ps_mats[li]

    g = _run_ga(h, ps_mats[2], ga_W1, ga_b1.reshape(1, 512),
                ga_W2, ga_b2.reshape(1, 1024))
    return _run_head(g, lin1_W, lin1_b.reshape(1, 512),
                     lin2_W, lin2_b.reshape(1, 256),
                     lin3_W, lin3_b.reshape(1, 40))


# speedup vs baseline: 1.2443x; 1.0090x over previous
import functools

import jax
import jax.numpy as jnp
from jax import lax
from jax.experimental import pallas as pl
from jax.experimental.pallas import tpu as pltpu
from jax.experimental.pallas import tpu_sc as plsc

B, P, NUM_CLASSES, K = 8, 1024, 40, 32
RADII = (0.2, 0.3, 0.4)
LEVELS = ((1024, 512, 0.2), (512, 256, 0.3), (256, 128, 0.4))
NEG_INF = float("-inf")
POS_INF = float("inf")


def _fps_planes(px, py, pz, S):
    Bb, Pl = px.shape
    lane_s = lax.broadcasted_iota(jnp.int32, (Bb, S), 1)
    lane_p = lax.broadcasted_iota(jnp.int32, (Bb, Pl), 1)

    def body(i, carry):
        mind, cx, cy, cz, sx, sy, sz = carry
        rec = lane_s == i
        sx = jnp.where(rec, cx, sx)
        sy = jnp.where(rec, cy, sy)
        sz = jnp.where(rec, cz, sz)
        dx = px - cx
        dy = py - cy
        dz = pz - cz
        d = dx * dx + dy * dy + dz * dz
        mind = jnp.minimum(mind, d)
        mx = jnp.max(mind, axis=1, keepdims=True)
        idx = jnp.min(jnp.where(mind == mx, lane_p, Pl), axis=1, keepdims=True)
        oh = lane_p == idx
        cx = jnp.sum(jnp.where(oh, px, 0.0), axis=1, keepdims=True)
        cy = jnp.sum(jnp.where(oh, py, 0.0), axis=1, keepdims=True)
        cz = jnp.sum(jnp.where(oh, pz, 0.0), axis=1, keepdims=True)
        return (mind, cx, cy, cz, sx, sy, sz)

    init = (
        jnp.full((Bb, Pl), POS_INF, jnp.float32),
        px[:, 0:1], py[:, 0:1], pz[:, 0:1],
        jnp.zeros((Bb, S), jnp.float32),
        jnp.zeros((Bb, S), jnp.float32),
        jnp.zeros((Bb, S), jnp.float32),
    )
    _, _, _, _, sx, sy, sz = lax.fori_loop(0, S, body, init)
    return sx, sy, sz


NG = 2
GB = B // NG


def _fps_kernel(planes_ref, ps1_ref, ps2_ref, ps3_ref):
    ps_refs = (ps1_ref, ps2_ref, ps3_ref)
    src = (planes_ref[0, 0], planes_ref[0, 1], planes_ref[0, 2])
    for li, (Pl, S, r) in enumerate(LEVELS):
        px, py, pz = src
        sx, sy, sz = _fps_planes(px, py, pz, S)
        ps_refs[li][0, 0] = sx
        ps_refs[li][0, 1] = sy
        ps_refs[li][0, 2] = sz
        src = (sx, sy, sz)


def _run_fps(planes):
    outs = [
        jax.ShapeDtypeStruct((NG, 3, GB, 512), jnp.float32),
        jax.ShapeDtypeStruct((NG, 3, GB, 256), jnp.float32),
        jax.ShapeDtypeStruct((NG, 3, GB, 128), jnp.float32),
    ]
    return pl.pallas_call(
        _fps_kernel,
        grid=(NG,),
        in_specs=[pl.BlockSpec((1, 3, GB, P), lambda i: (i, 0, 0, 0))],
        out_specs=[
            pl.BlockSpec((1, 3, GB, 512), lambda i: (i, 0, 0, 0)),
            pl.BlockSpec((1, 3, GB, 256), lambda i: (i, 0, 0, 0)),
            pl.BlockSpec((1, 3, GB, 128), lambda i: (i, 0, 0, 0)),
        ],
        out_shape=outs,
        compiler_params=pltpu.CompilerParams(
            dimension_semantics=("parallel",)),
    )(planes.reshape(NG, GB, P, 3).transpose(0, 3, 1, 2))


def _ext_kernel(S, Pl, r, qs_ref, pts_ref, n_ref, d2_ref):
    Bb = qs_ref.shape[2]
    px, py, pz = pts_ref[0, 0], pts_ref[0, 1], pts_ref[0, 2]
    sx, sy, sz = qs_ref[0, 0], qs_ref[0, 1], qs_ref[0, 2]
    dx = sx[:, None, :] - px[:, :, None]
    dy = sy[:, None, :] - py[:, :, None]
    dz = sz[:, None, :] - pz[:, :, None]
    d2 = dx * dx + dy * dy + dz * dz
    d2_ref[...] = jnp.where(d2 <= r * r, d2, POS_INF)

    sub_p = lax.broadcasted_iota(jnp.int32, (Bb, Pl, S), 1)
    sub_k = lax.broadcasted_iota(jnp.int32, (Bb, K, S), 1)

    RU = 4

    def round_body(kk, idx0):
        d2v = d2_ref[...]
        nacc = n_ref[0]
        for u in range(RU):
            m = jnp.min(d2v, axis=1, keepdims=True)
            idxv = jnp.min(jnp.where(d2v == m, sub_p, Pl), axis=1,
                           keepdims=True)
            if u == 0:
                idx0 = jnp.where(kk == 0, idxv, idx0)
            idxf = jnp.where(m < POS_INF, idxv, idx0)
            nacc = jnp.where(sub_k == kk * RU + u, idxf, nacc)
            d2v = jnp.where(sub_p == idxv, POS_INF, d2v)
        n_ref[0] = nacc
        d2_ref[...] = d2v
        return idx0

    lax.fori_loop(0, K // RU, round_body, jnp.zeros((Bb, 1, S), jnp.int32))


def _run_ext(qs, pts, S, Pl, r):
    return pl.pallas_call(
        functools.partial(_ext_kernel, S, Pl, r),
        grid=(NG,),
        in_specs=[
            pl.BlockSpec((1, 3, GB, S), lambda i: (i, 0, 0, 0)),
            pl.BlockSpec((1, 3, GB, Pl), lambda i: (i, 0, 0, 0)),
        ],
        out_specs=pl.BlockSpec((1, GB, K, S), lambda i: (i, 0, 0, 0)),
        out_shape=jax.ShapeDtypeStruct((NG, GB, K, S), jnp.int32),
        scratch_shapes=[pltpu.VMEM((GB, Pl, S), jnp.float32)],
        compiler_params=pltpu.CompilerParams(
            dimension_semantics=("parallel",)),
    )(qs, pts)


def _table_kernel(C, h_ref, pos_ref, ps_ref, W1_ref, b1_ref, t_ref, q_ref):
    if C == 3:
        hv = h_ref[0]
        t = (hv[:, 0:1] * W1_ref[0:1, :]
             + hv[:, 1:2] * W1_ref[1:2, :]
             + hv[:, 2:3] * W1_ref[2:3, :])
    else:
        t = jnp.dot(h_ref[0], W1_ref[0:C, :],
                    preferred_element_type=jnp.float32)
    pv = pos_ref[0]
    t = t + (pv[:, 0:1] * W1_ref[C:C + 1, :]
             + pv[:, 1:2] * W1_ref[C + 1:C + 2, :]
             + pv[:, 2:3] * W1_ref[C + 2:C + 3, :])
    t_ref[0] = t + b1_ref[0:1, :]
    sv = ps_ref[0]
    q_ref[0] = (sv[:, 0:1] * W1_ref[C:C + 1, :]
                + sv[:, 1:2] * W1_ref[C + 1:C + 2, :]
                + sv[:, 2:3] * W1_ref[C + 2:C + 3, :])


def _run_table(h, pos_s_prev, ps, W1, b1, C, Pl, S, d1):
    return pl.pallas_call(
        functools.partial(_table_kernel, C),
        grid=(B,),
        in_specs=[
            pl.BlockSpec((1, Pl, C), lambda b: (b, 0, 0)),
            pl.BlockSpec((1, Pl, 3), lambda b: (b, 0, 0)),
            pl.BlockSpec((1, S, 3), lambda b: (b, 0, 0)),
            pl.BlockSpec((C + 3, d1), lambda b: (0, 0)),
            pl.BlockSpec((1, d1), lambda b: (0, 0)),
        ],
        out_specs=[
            pl.BlockSpec((1, Pl, d1), lambda b: (b, 0, 0)),
            pl.BlockSpec((1, S, d1), lambda b: (b, 0, 0)),
        ],
        out_shape=[
            jax.ShapeDtypeStruct((B, Pl, d1), jnp.float32),
            jax.ShapeDtypeStruct((B, S, d1), jnp.float32),
        ],
        compiler_params=pltpu.CompilerParams(
            dimension_semantics=("parallel",)),
    )(h, pos_s_prev, ps, W1, b1)


def _sc_gather(table, idx, N, D):
    NC, NS = 2, 16
    NW = NC * NS
    n = N // NW
    CH = 128
    nch = n // CH
    idx2 = idx.reshape(NW * nch, CH)
    mesh = plsc.VectorSubcoreMesh(core_axis_name="c", subcore_axis_name="s")

    UB = min(nch, 8)
    nblk = nch // UB

    @functools.partial(
        pl.kernel,
        out_type=jax.ShapeDtypeStruct((N, D), jnp.float32),
        mesh=mesh,
        scratch_types=[
            pltpu.VMEM((nch, CH), jnp.int32),
            pltpu.VMEM((CH,), jnp.int32),
            pltpu.VMEM((CH,), jnp.int32),
            pltpu.VMEM((CH, D), jnp.float32),
            pltpu.VMEM((CH, D), jnp.float32),
            pltpu.SemaphoreType.DMA,
            pltpu.SemaphoreType.DMA,
            pltpu.SemaphoreType.DMA,
            pltpu.SemaphoreType.DMA,
        ],
    )
    def gk(table_hbm, idx_hbm, out_hbm, idx_all, idx0, idx1, buf0, buf1,
           semg0, semg1, semo0, semo1):
        wid = lax.axis_index("s") * NC + lax.axis_index("c")
        base = wid * n
        pltpu.sync_copy(idx_hbm.at[pl.ds(wid * nch, nch)], idx_all)
        idxs = (idx0, idx1)
        bufs = (buf0, buf1)
        semg = (semg0, semg1)
        semo = (semo0, semo1)

        def block(blk, carry):
            gh = [None, None]
            oh = [None, None]
            for j2 in range(UB):
                p = j2 % 2
                j = blk * UB + j2
                for i in range(CH // 16):
                    idxs[p][pl.ds(i * 16, 16)] = idx_all[j, pl.ds(i * 16, 16)]
                if oh[p] is not None:
                    oh[p].wait()
                gh[p] = pltpu.async_copy(table_hbm.at[idxs[p]], bufs[p],
                                         semg[p])
                if gh[1 - p] is not None:
                    gh[1 - p].wait()
                    oh[1 - p] = pltpu.async_copy(
                        bufs[1 - p],
                        out_hbm.at[pl.ds(base + (j - 1) * CH, CH)],
                        semo[1 - p])
                    gh[1 - p] = None
            pl_last = (UB - 1) % 2
            gh[pl_last].wait()
            oh[pl_last] = pltpu.async_copy(
                bufs[pl_last],
                out_hbm.at[pl.ds(base + (blk * UB + UB - 1) * CH, CH)],
                semo[pl_last])
            for p in range(2):
                if oh[p] is not None:
                    oh[p].wait()
            return carry

        lax.fori_loop(0, nblk, block, 0)

    return gk(table, idx2)


def _agg_kernel(g_ref, q_ref, W2_ref, b2_ref, out_ref):
    qv = q_ref[0]
    S, d1 = qv.shape
    d2 = W2_ref.shape[1]
    m = jnp.maximum(g_ref[0] - qv[None, :, :], 0.0)
    z = jnp.dot(m.reshape(K * S, d1), W2_ref[:, :],
                preferred_element_type=jnp.float32).reshape(K, S, d2)
    zs = [z[k] for k in range(K)]
    while len(zs) > 1:
        zs = [jnp.maximum(zs[i], zs[i + 1]) for i in range(0, len(zs), 2)]
    out_ref[0] = jnp.maximum(zs[0] + b2_ref[0:1, :], 0.0)


def _run_agg(g, q, W2, b2, S, d1, d2):
    return pl.pallas_call(
        _agg_kernel,
        grid=(B,),
        in_specs=[
            pl.BlockSpec((1, K, S, d1), lambda b: (b, 0, 0, 0)),
            pl.BlockSpec((1, S, d1), lambda b: (b, 0, 0)),
            pl.BlockSpec((d1, d2), lambda b: (0, 0)),
            pl.BlockSpec((1, d2), lambda b: (0, 0)),
        ],
        out_specs=pl.BlockSpec((1, S, d2), lambda b: (b, 0, 0)),
        out_shape=jax.ShapeDtypeStruct((B, S, d2), jnp.float32),
        compiler_params=pltpu.CompilerParams(
            dimension_semantics=("parallel",)),
    )(g, q, W2, b2)


def _ga_kernel(h_ref, ps_ref, W1_ref, b1_ref, W2_ref, b2_ref, out_ref):
    C = h_ref.shape[2]
    t = jnp.dot(h_ref[0], W1_ref[0:C, :], preferred_element_type=jnp.float32)
    pv = ps_ref[0]
    t = t + (pv[:, 0:1] * W1_ref[C:C + 1, :]
             + pv[:, 1:2] * W1_ref[C + 1:C + 2, :]
             + pv[:, 2:3] * W1_ref[C + 2:C + 3, :])
    t = jnp.maximum(t + b1_ref[0:1, :], 0.0)
    z = jnp.dot(t, W2_ref[:, :], preferred_element_type=jnp.float32)
    z = jnp.maximum(z + b2_ref[0:1, :], 0.0)
    out_ref[0] = jnp.max(z, axis=0, keepdims=True)


def _run_ga(h3, ps3, W1, b1, W2, b2):
    return pl.pallas_call(
        _ga_kernel,
        grid=(B,),
        in_specs=[
            pl.BlockSpec((1, 128, 256), lambda b: (b, 0, 0)),
            pl.BlockSpec((1, 128, 3), lambda b: (b, 0, 0)),
            pl.BlockSpec((259, 512), lambda b: (0, 0)),
            pl.BlockSpec((1, 512), lambda b: (0, 0)),
            pl.BlockSpec((512, 1024), lambda b: (0, 0)),
            pl.BlockSpec((1, 1024), lambda b: (0, 0)),
        ],
        out_specs=pl.BlockSpec((1, 1, 1024), lambda b: (b, 0, 0)),
        out_shape=jax.ShapeDtypeStruct((B, 1, 1024), jnp.float32),
        compiler_params=pltpu.CompilerParams(
            dimension_semantics=("parallel",)),
    )(h3, ps3, W1, b1, W2, b2).reshape(B, 1024)


def _head_kernel(g_ref, W1_ref, b1_ref, W2_ref, b2_ref, W3_ref, b3_ref,
                 out_ref):
    h = jnp.dot(g_ref[:, :], W1_ref[:, :], preferred_element_type=jnp.float32)
    h = jnp.maximum(h + b1_ref[0:1, :], 0.0)
    h = jnp.dot(h, W2_ref[:, :], preferred_element_type=jnp.float32)
    h = jnp.maximum(h + b2_ref[0:1, :], 0.0)
    h = jnp.dot(h, W3_ref[:, :], preferred_element_type=jnp.float32)
    out_ref[:, :] = h + b3_ref[0:1, :]


def _run_head(g, lin1_W, lin1_b, lin2_W, lin2_b, lin3_W, lin3_b):
    return pl.pallas_call(
        _head_kernel,
        out_shape=jax.ShapeDtypeStruct((B, NUM_CLASSES), jnp.float32),
    )(g, lin1_W, lin1_b, lin2_W, lin2_b, lin3_W, lin3_b)


def kernel(x, pos, batch, sa1_W1, sa1_b1, sa1_W2, sa1_b2, sa2_W1, sa2_b1,
           sa2_W2, sa2_b2, sa3_W1, sa3_b1, sa3_W2, sa3_b2, ga_W1, ga_b1,
           ga_W2, ga_b2, lin1_W, lin1_b, lin2_W, lin2_b, lin3_W, lin3_b):
    pos_bp3 = pos.reshape(B, P, 3)
    pos4 = pos_bp3.reshape(NG, GB, P, 3).transpose(0, 3, 1, 2)
    ps1, ps2, ps3 = _run_fps(pos_bp3)

    ps_mats = [p.transpose(0, 2, 3, 1).reshape(B, -1, 3)
               for p in (ps1, ps2, ps3)]

    h = x.reshape(B, P, 3)
    pos_prev = pos_bp3
    level_ws = (
        (jnp.pad(sa1_W1, ((0, 0), (0, 96))), jnp.pad(sa1_b1, (0, 96)),
         jnp.pad(sa1_W2, ((0, 96), (0, 0))), sa1_b2),
        (sa2_W1, sa2_b1, sa2_W2, sa2_b2),
        (sa3_W1, sa3_b1, sa3_W2, sa3_b2),
    )
    qs_list = (ps1, ps2, ps3)
    pts_list = (pos4, ps1, ps2)
    nidx = _run_ext(qs_list[0], pts_list[0], LEVELS[0][1], LEVELS[0][0],
                    LEVELS[0][2])
    for li, (Pl, S, _r) in enumerate(LEVELS):
        W1, b1, W2, b2 = level_ws[li]
        C = h.shape[2]
        d1 = W1.shape[1]
        d2 = W2.shape[1]
        t, q = _run_table(h, pos_prev, ps_mats[li], W1, b1.reshape(1, d1),
                          C, Pl, S, d1)
        gidx = (nidx.reshape(B, K, S)
                + (jnp.arange(B, dtype=jnp.int32) * Pl)[:, None, None])
        N = B * K * S
        g = _sc_gather(t.reshape(B * Pl, d1), gidx.reshape(-1), N, d1)
        if li + 1 < len(LEVELS):
            nl = LEVELS[li + 1]
            nidx = _run_ext(qs_list[li + 1], pts_list[li + 1], nl[1], nl[0],
                            nl[2])
        h = _run_agg(g.reshape(B, K, S, d1), q, W2, b2.reshape(1, d2),
                     S, d1, d2)
        pos_prev = ps_mats[li]

    g = _run_ga(h, ps_mats[2], ga_W1, ga_b1.reshape(1, 512),
                ga_W2, ga_b2.reshape(1, 1024))
    return _run_head(g, lin1_W, lin1_b.reshape(1, 512),
                     lin2_W, lin2_b.reshape(1, 256),
                     lin3_W, lin3_b.reshape(1, 40))
